# Initial kernel scaffold; baseline (speedup 1.0000x reference)
import jax
import jax.numpy as jnp
from jax.experimental import pallas as pl


def kernel(x, token_embedding_weight):
    raise NotImplementedError("write your pallas kernel here")


# speedup vs baseline: 2.2647x; 2.2647x over previous
import functools

import jax
import jax.numpy as jnp
from jax import lax
from jax.experimental import pallas as pl
from jax.experimental.pallas import tpu as pltpu
from jax.experimental.pallas import tpu_sc as plsc

_MAX_SEQUENCE_LENGTH = 10000

_NC, _NS, _L = 2, 16, 16
_NW = _NC * _NS


def _positional_encoding(seq_len, d_model):
    position = jnp.arange(seq_len, dtype=jnp.float32).reshape(seq_len, 1)
    dim = jnp.floor_divide(jnp.linspace(0.0, d_model - 1, d_model), 2.0) * 2.0
    dim = dim / d_model
    denom = jnp.power(jnp.float32(_MAX_SEQUENCE_LENGTH), dim)
    angles = position / denom
    col = jnp.arange(d_model)
    return jnp.where(col % 2 == 0, jnp.sin(angles), jnp.cos(angles)).astype(jnp.float32)


@functools.cache
def _build(B, S, D):
    N = B * S
    rows_per_w = N // _NW
    chunk = 128
    assert N % _NW == 0 and rows_per_w % chunk == 0 and chunk % 8 == 0
    assert rows_per_w % S == 0
    assert D % _L == 0
    nchunk = rows_per_w // chunk

    mesh = plsc.VectorSubcoreMesh(core_axis_name="c", subcore_axis_name="s")

    @functools.partial(
        pl.kernel,
        mesh=mesh,
        out_type=jax.ShapeDtypeStruct((N, D), jnp.float32),
        scratch_types=[
            pltpu.VMEM((rows_per_w,), jnp.int32),
            pltpu.VMEM((2 * S, D), jnp.float32),
            pltpu.VMEM((chunk, D), jnp.float32),
            pltpu.SemaphoreType.DMA,
        ],
    )
    def emb(idx_hbm, table_hbm, pe_hbm, out_hbm, idx_v, pe_v, rows_v, sem):
        wid = lax.axis_index("s") * _NC + lax.axis_index("c")
        base = wid * rows_per_w
        pltpu.sync_copy(idx_hbm.at[pl.ds(base, rows_per_w)], idx_v)
        pltpu.sync_copy(pe_hbm, pe_v)

        def chunk_body(i, carry):
            cb = i * chunk
            s0 = lax.rem(cb, S)
            pltpu.async_copy(
                table_hbm.at[idx_v.at[pl.ds(cb, chunk)]], rows_v, sem
            ).wait()

            def row_body(r, c2):
                for g in range(D // _L):
                    pe_vec = pe_v[s0 + r, pl.ds(g * _L, _L)]
                    plsc.addupdate(rows_v.at[r, pl.ds(g * _L, _L)], pe_vec)
                return c2

            lax.fori_loop(0, chunk, row_body, 0)
            pltpu.sync_copy(rows_v, out_hbm.at[pl.ds(base + cb, chunk)])
            return carry

        lax.fori_loop(0, nchunk, chunk_body, 0)

    return emb


def kernel(x, token_embedding_weight):
    B, S = x.shape
    _, D = token_embedding_weight.shape
    pe = _positional_encoding(S, D)
    pe2 = jnp.concatenate([pe, pe], axis=0)
    idx = x.reshape(-1).astype(jnp.int32)
    out = _build(B, S, D)(idx, token_embedding_weight, pe2)
    return out.reshape(B, S, D)


# speedup vs baseline: 3.5080x; 1.5490x over previous
import functools

import jax
import jax.numpy as jnp
from jax import lax
from jax.experimental import pallas as pl
from jax.experimental.pallas import tpu as pltpu
from jax.experimental.pallas import tpu_sc as plsc

_MAX_SEQUENCE_LENGTH = 10000

_NC, _NS, _L = 2, 16, 16
_NW = _NC * _NS
_NBUF = 3


def _positional_encoding(seq_len, d_model):
    position = jnp.arange(seq_len, dtype=jnp.float32).reshape(seq_len, 1)
    dim = jnp.floor_divide(jnp.linspace(0.0, d_model - 1, d_model), 2.0) * 2.0
    dim = dim / d_model
    denom = jnp.power(jnp.float32(_MAX_SEQUENCE_LENGTH), dim)
    angles = position / denom
    col = jnp.arange(d_model)
    return jnp.where(col % 2 == 0, jnp.sin(angles), jnp.cos(angles)).astype(jnp.float32)


@functools.cache
def _build(B, S, D):
    N = B * S
    rows_per_w = N // _NW
    chunk = 128
    assert N % _NW == 0 and rows_per_w % chunk == 0 and chunk % 8 == 0
    assert rows_per_w % S == 0
    assert D % _L == 0
    nchunk = rows_per_w // chunk
    assert nchunk % 3 != 1 and nchunk >= 2 * _NBUF

    mesh = plsc.VectorSubcoreMesh(core_axis_name="c", subcore_axis_name="s")

    rows_t = pltpu.VMEM((chunk, D), jnp.float32)

    @functools.partial(
        pl.kernel,
        mesh=mesh,
        out_type=jax.ShapeDtypeStruct((N, D), jnp.float32),
        scratch_types=[
            pltpu.VMEM((rows_per_w,), jnp.int32),
            pltpu.VMEM((2 * S, D), jnp.float32),
            rows_t, rows_t, rows_t,
            pltpu.SemaphoreType.DMA, pltpu.SemaphoreType.DMA,
            pltpu.SemaphoreType.DMA, pltpu.SemaphoreType.DMA,
            pltpu.SemaphoreType.DMA, pltpu.SemaphoreType.DMA,
        ],
    )
    def emb(idx_hbm, table_hbm, pe_hbm, out_hbm,
            idx_v, pe_v, rb0, rb1, rb2, g0, g1, g2, s0, s1, s2):
        wid = lax.axis_index("s") * _NC + lax.axis_index("c")
        base = wid * rows_per_w
        pltpu.sync_copy(idx_hbm.at[pl.ds(base, rows_per_w)], idx_v)
        pltpu.sync_copy(pe_hbm, pe_v)

        bufs = (rb0, rb1, rb2)
        gsems = (g0, g1, g2)
        ssems = (s0, s1, s2)

        def gcopy(k, b):
            return pltpu.make_async_copy(
                table_hbm.at[idx_v.at[pl.ds(k * chunk, chunk)]], bufs[b], gsems[b])

        def scopy(k, b):
            return pltpu.make_async_copy(
                bufs[b], out_hbm.at[pl.ds(base + k * chunk, chunk)], ssems[b])

        def add_pe(k, b):
            pe0 = lax.rem(k * chunk, S)
            buf = bufs[b]

            def row_body(q, carry):
                r = q * 4
                for u in range(4):
                    for g in range(D // _L):
                        plsc.addupdate(
                            buf.at[r + u, pl.ds(g * _L, _L)],
                            pe_v[pe0 + r + u, pl.ds(g * _L, _L)])
                return carry

            lax.fori_loop(0, chunk // 4, row_body, 0)

        def step(j, *, swait_prev=True, gstart_ahead=True):
            b = j % 3 if isinstance(j, int) else None
            assert b is not None
            gcopy(j, b).wait()
            add_pe(j, b)
            scopy(j, b).start()
            if swait_prev:
                scopy(j - 1, (j - 1) % 3).wait()
            if gstart_ahead:
                gcopy(j + 2, (j + 2) % 3).start()

        def dyn_step(j, b):
            gcopy(j, b).wait()
            add_pe(j, b)
            scopy(j, b).start()
            scopy(j - 1, (b - 1) % 3).wait()
            gcopy(j + 2, (b + 2) % 3).start()

        gcopy(0, 0).start()
        gcopy(1, 1).start()
        step(0, swait_prev=False)
        step(1)
        step(2)

        def mid(p, carry):
            j = p * 3
            for u in range(3):
                dyn_step(j + u, u)
            return carry

        lax.fori_loop(1, nchunk // 3, mid, 0)

        step(nchunk - 2, gstart_ahead=False)
        step(nchunk - 1, gstart_ahead=False)
        scopy(nchunk - 1, (nchunk - 1) % 3).wait()

    return emb


def kernel(x, token_embedding_weight):
    B, S = x.shape
    _, D = token_embedding_weight.shape
    pe = _positional_encoding(S, D)
    pe2 = jnp.concatenate([pe, pe], axis=0)
    idx = x.reshape(-1).astype(jnp.int32)
    out = _build(B, S, D)(idx, token_embedding_weight, pe2)
    return out.reshape(B, S, D)


# speedup vs baseline: 9.2674x; 2.6418x over previous
import functools

import jax
import jax.numpy as jnp
from jax import lax
from jax.experimental import pallas as pl
from jax.experimental.pallas import tpu as pltpu
from jax.experimental.pallas import tpu_sc as plsc

_MAX_SEQUENCE_LENGTH = 10000

_NC, _NS, _L = 2, 16, 16
_NW = _NC * _NS


def _positional_encoding(seq_len, d_model):
    position = jnp.arange(seq_len, dtype=jnp.float32).reshape(seq_len, 1)
    dim = jnp.floor_divide(jnp.linspace(0.0, d_model - 1, d_model), 2.0) * 2.0
    dim = dim / d_model
    denom = jnp.power(jnp.float32(_MAX_SEQUENCE_LENGTH), dim)
    angles = position / denom
    col = jnp.arange(d_model)
    return jnp.where(col % 2 == 0, jnp.sin(angles), jnp.cos(angles)).astype(jnp.float32)


@functools.cache
def _build(B, S, D):
    N = B * S
    chunk = B // _NW
    nchunk = S
    rows_per_w = chunk * nchunk
    assert B % _NW == 0 and chunk <= 128 and chunk % _L == 0
    assert D % _L == 0
    assert nchunk % 3 == 2 and nchunk >= 8

    mesh = plsc.VectorSubcoreMesh(core_axis_name="c", subcore_axis_name="s")

    rows_t = pltpu.VMEM((chunk, D), jnp.float32)
    oidx_t = pltpu.VMEM((chunk,), jnp.int32)

    @functools.partial(
        pl.kernel,
        mesh=mesh,
        out_type=jax.ShapeDtypeStruct((N, D), jnp.float32),
        scratch_types=[
            pltpu.VMEM((rows_per_w,), jnp.int32),
            pltpu.VMEM((S, D), jnp.float32),
            rows_t, rows_t, rows_t,
            oidx_t, oidx_t, oidx_t,
            pltpu.SemaphoreType.DMA, pltpu.SemaphoreType.DMA,
            pltpu.SemaphoreType.DMA, pltpu.SemaphoreType.DMA,
            pltpu.SemaphoreType.DMA, pltpu.SemaphoreType.DMA,
        ],
    )
    def emb(idx_hbm, table_hbm, pe_hbm, out_hbm,
            idx_v, pe_v, rb0, rb1, rb2, ox0, ox1, ox2, g0, g1, g2, s0, s1, s2):
        wid = lax.axis_index("s") * _NC + lax.axis_index("c")
        base = wid * rows_per_w
        pltpu.sync_copy(idx_hbm.at[pl.ds(base, rows_per_w)], idx_v)
        pltpu.sync_copy(pe_hbm, pe_v)

        bufs = (rb0, rb1, rb2)
        oxs = (ox0, ox1, ox2)
        gsems = (g0, g1, g2)
        ssems = (s0, s1, s2)
        jbase = wid * chunk * S

        def gcopy(c, b):
            return pltpu.make_async_copy(
                table_hbm.at[idx_v.at[pl.ds(c * chunk, chunk)]], bufs[b], gsems[b])

        def scopy(b):
            return pltpu.make_async_copy(bufs[b], out_hbm.at[oxs[b]], ssems[b])

        def add_pe_and_oidx(c, b):
            buf, ox = bufs[b], oxs[b]
            obase = jbase + c
            for g in range(chunk // _L):
                lane = lax.iota(jnp.int32, _L) + (g * _L)
                ox[pl.ds(g * _L, _L)] = lane * S + obase
            pe_regs = [pe_v[c, pl.ds(g * _L, _L)] for g in range(D // _L)]

            @plsc.parallel_loop(0, chunk, step=1, unroll=2)
            def _(j):
                for g in range(D // _L):
                    plsc.addupdate(buf.at[j, pl.ds(g * _L, _L)], pe_regs[g])

        def step(c, b, *, swait_prev=True, gstart_ahead=True):
            gcopy(c, b).wait()
            add_pe_and_oidx(c, b)
            scopy(b).start()
            if swait_prev:
                scopy((b - 1) % 3).wait()
            if gstart_ahead:
                gcopy(c + 2, (b + 2) % 3).start()

        gcopy(0, 0).start()
        gcopy(1, 1).start()
        step(0, 0, swait_prev=False)
        step(1, 1)
        step(2, 2)

        def mid(p, carry):
            c = p * 3
            for u in range(3):
                step(c + u, u)
            return carry

        lax.fori_loop(1, nchunk // 3, mid, 0)

        step(nchunk - 2, 0, gstart_ahead=False)
        step(nchunk - 1, 1, gstart_ahead=False)
        scopy(1).wait()

    return emb


def kernel(x, token_embedding_weight):
    B, S = x.shape
    _, D = token_embedding_weight.shape
    pe = _positional_encoding(S, D)
    idx = (x.reshape(_NW, B // _NW, S)
             .transpose(0, 2, 1)
             .reshape(-1)
             .astype(jnp.int32))
    out = _build(B, S, D)(idx, token_embedding_weight, pe)
    return out.reshape(B, S, D)
